# Initial kernel scaffold; baseline (speedup 1.0000x reference)
import jax
import jax.numpy as jnp
from jax.experimental import pallas as pl


def kernel(x, id_table, pos_table, ln_gamma, ln_beta):
    raise NotImplementedError("write your pallas kernel here")


# speedup vs baseline: 2.0307x; 2.0307x over previous
import jax
import jax.numpy as jnp
from jax import lax
from jax.experimental import pallas as pl
from jax.experimental.pallas import tpu as pltpu
from jax.experimental.pallas import tpu_sc as plsc

B = 4096
S = 200
D = 128
N = B * S
NC = 2
NS = 16
NW = NC * NS
ROWS_W = N // NW
CH = 128
NCH = ROWS_W // CH
L = 16
KV = D // L
EPS = 1e-12


def _ln_body(x_hbm, tab_hbm, pos_hbm, g_hbm, b_hbm, out_hbm,
             idx_v, rows_v, pos_v, g_v, b_v, sem):
    wid = lax.axis_index("s") * NC + lax.axis_index("c")
    base_w = wid * ROWS_W

    pltpu.sync_copy(pos_hbm, pos_v)
    pltpu.sync_copy(g_hbm, g_v)
    pltpu.sync_copy(b_hbm, b_v)
    g = [g_v[pl.ds(k * L, L)] for k in range(KV)]
    bta = [b_v[pl.ds(k * L, L)] for k in range(KV)]

    lanes = lax.iota(jnp.int32, L)
    bfly = [jnp.bitwise_xor(lanes, sh).reshape(L, 1) for sh in (1, 2, 4, 8)]
    gdn = lax.GatherDimensionNumbers(
        offset_dims=(), collapsed_slice_dims=(0,), start_index_map=(0,))

    def allreduce_sum(v):
        for idx in bfly:
            v = v + lax.gather(v, idx, gdn, (1,),
                               mode=lax.GatherScatterMode.PROMISE_IN_BOUNDS)
        return v

    def chunk(c, carry):
        base = base_w + c * CH
        pltpu.sync_copy(x_hbm.at[pl.ds(base, CH)], idx_v)
        pltpu.async_copy(tab_hbm.at[idx_v], rows_v, sem).wait()

        def row(r, rcarry):
            p = lax.rem(c * CH + r, S)
            t = []
            for k in range(KV):
                tk = rows_v[r, pl.ds(k * L, L)] + pos_v[p, pl.ds(k * L, L)]
                t.append(tk)
            acc = t[0]
            acc2 = t[0] * t[0]
            for k in range(1, KV):
                acc = acc + t[k]
                acc2 = acc2 + t[k] * t[k]
            mean = allreduce_sum(acc) * (1.0 / D)
            ex2 = allreduce_sum(acc2) * (1.0 / D)
            var = ex2 - mean * mean + EPS
            xi = lax.bitcast_convert_type(var, jnp.int32)
            yi = jnp.int32(0x5F3759DF) - lax.shift_right_logical(xi, 1)
            y = lax.bitcast_convert_type(yi, jnp.float32)
            h = var * 0.5
            for _ in range(3):
                y = y * (1.5 - h * y * y)
            for k in range(KV):
                u = (t[k] - mean) * y * g[k] + bta[k]
                rows_v[r, pl.ds(k * L, L)] = u
            return rcarry

        lax.fori_loop(0, CH, row, 0)
        pltpu.sync_copy(rows_v, out_hbm.at[pl.ds(base, CH)])
        return carry

    lax.fori_loop(0, NCH, chunk, 0)


def kernel(x, id_table, pos_table, ln_gamma, ln_beta):
    xf = x.reshape(N)
    mesh = plsc.VectorSubcoreMesh(core_axis_name="c", subcore_axis_name="s")
    run = pl.kernel(
        _ln_body,
        mesh=mesh,
        out_type=jax.ShapeDtypeStruct((N, D), jnp.float32),
        scratch_types=[
            pltpu.VMEM((CH,), jnp.int32),
            pltpu.VMEM((CH, D), jnp.float32),
            pltpu.VMEM((S, D), jnp.float32),
            pltpu.VMEM((D,), jnp.float32),
            pltpu.VMEM((D,), jnp.float32),
            pltpu.SemaphoreType.DMA,
        ],
    )
    out = run(xf, id_table, pos_table, ln_gamma, ln_beta)
    return out.reshape(B, S, D)


# speedup vs baseline: 2.6207x; 1.2906x over previous
import jax
import jax.numpy as jnp
from jax import lax
from jax.experimental import pallas as pl
from jax.experimental.pallas import tpu as pltpu
from jax.experimental.pallas import tpu_sc as plsc

B = 4096
S = 200
D = 128
N = B * S
NC = 2
NS = 16
NW = NC * NS
ROWS_W = N // NW
CH = 128
NCH = ROWS_W // CH
NBUF = 4
L = 16
KV = D // L
EPS = 1e-12


def _ln_body(x_hbm, tab_hbm, pos_hbm, g_hbm, b_hbm, out_hbm,
             idx0, idx1, idx2, idx3, rows0, rows1, rows2, rows3,
             pos_v, g_v, b_v,
             gs0, gs1, gs2, gs3, ws0, ws1, ws2, ws3):
    idx_v = [idx0, idx1, idx2, idx3]
    rows_v = [rows0, rows1, rows2, rows3]
    gsem = [gs0, gs1, gs2, gs3]
    wsem = [ws0, ws1, ws2, ws3]

    wid = lax.axis_index("s") * NC + lax.axis_index("c")
    base_w = wid * ROWS_W

    pltpu.sync_copy(pos_hbm, pos_v)
    pltpu.sync_copy(g_hbm, g_v)
    pltpu.sync_copy(b_hbm, b_v)
    g = [g_v[pl.ds(k * L, L)] for k in range(KV)]
    bta = [b_v[pl.ds(k * L, L)] for k in range(KV)]

    lanes = lax.iota(jnp.int32, L)
    bfly = [jnp.bitwise_xor(lanes, sh).reshape(L, 1) for sh in (1, 2, 4, 8)]
    gdn = lax.GatherDimensionNumbers(
        offset_dims=(), collapsed_slice_dims=(0,), start_index_map=(0,))

    def allreduce_sum(v):
        for idx in bfly:
            v = v + lax.gather(v, idx, gdn, (1,),
                               mode=lax.GatherScatterMode.PROMISE_IN_BOUNDS)
        return v

    def start_gather(cc, b):
        pltpu.sync_copy(x_hbm.at[pl.ds(base_w + cc * CH, CH)], idx_v[b])
        pltpu.async_copy(tab_hbm.at[idx_v[b]], rows_v[b], gsem[b])

    def compute_chunk(cc, b):
        buf = rows_v[b]

        def row(r, rcarry):
            p = lax.rem(cc * CH + r, S)
            t = []
            for k in range(KV):
                tk = buf[r, pl.ds(k * L, L)] + pos_v[p, pl.ds(k * L, L)]
                t.append(tk)
            acc = t[0]
            acc2 = t[0] * t[0]
            for k in range(1, KV):
                acc = acc + t[k]
                acc2 = acc2 + t[k] * t[k]
            mean = allreduce_sum(acc) * (1.0 / D)
            ex2 = allreduce_sum(acc2) * (1.0 / D)
            var = ex2 - mean * mean + EPS
            xi = lax.bitcast_convert_type(var, jnp.int32)
            yi = jnp.int32(0x5F3759DF) - lax.shift_right_logical(xi, 1)
            y = lax.bitcast_convert_type(yi, jnp.float32)
            h = var * 0.5
            for _ in range(3):
                y = y * (1.5 - h * y * y)
            for k in range(KV):
                u = (t[k] - mean) * y * g[k] + bta[k]
                buf[r, pl.ds(k * L, L)] = u
            return rcarry

        lax.fori_loop(0, CH, row, 0, unroll=2)

    for b in range(NBUF - 1):
        start_gather(b, b)

    def ring(i, carry):
        for b in range(NBUF):
            cc = i * NBUF + b
            pb = (b - 1) % NBUF
            pltpu.make_async_copy(rows_v[b], out_hbm.at[pl.ds(0, CH)],
                                  gsem[b]).wait()
            compute_chunk(cc, b)
            pltpu.async_copy(rows_v[b],
                             out_hbm.at[pl.ds(base_w + cc * CH, CH)], wsem[b])

            @pl.when(cc >= 1)
            def _wait_prev_write():
                pltpu.make_async_copy(
                    rows_v[pb], out_hbm.at[pl.ds(0, CH)], wsem[pb]).wait()

            @pl.when(cc + NBUF - 1 < NCH)
            def _prefetch():
                start_gather(cc + NBUF - 1, pb)
        return carry

    lax.fori_loop(0, NCH // NBUF, ring, 0)
    pltpu.make_async_copy(rows_v[(NCH - 1) % NBUF], out_hbm.at[pl.ds(0, CH)],
                          wsem[(NCH - 1) % NBUF]).wait()


def kernel(x, id_table, pos_table, ln_gamma, ln_beta):
    xf = x.reshape(N)
    mesh = plsc.VectorSubcoreMesh(core_axis_name="c", subcore_axis_name="s")
    run = pl.kernel(
        _ln_body,
        mesh=mesh,
        out_type=jax.ShapeDtypeStruct((N, D), jnp.float32),
        scratch_types=(
            [pltpu.VMEM((CH,), jnp.int32) for _ in range(NBUF)]
            + [pltpu.VMEM((CH, D), jnp.float32) for _ in range(NBUF)]
            + [pltpu.VMEM((S, D), jnp.float32),
               pltpu.VMEM((D,), jnp.float32),
               pltpu.VMEM((D,), jnp.float32)]
            + [pltpu.SemaphoreType.DMA for _ in range(2 * NBUF)]
        ),
    )
    out = run(xf, id_table, pos_table, ln_gamma, ln_beta)
    return out.reshape(B, S, D)


# speedup vs baseline: 2.8768x; 1.0977x over previous
import jax
import jax.numpy as jnp
from jax import lax
from jax.experimental import pallas as pl
from jax.experimental.pallas import tpu as pltpu
from jax.experimental.pallas import tpu_sc as plsc

B = 4096
S = 200
D = 128
N = B * S
NC = 2
NS = 16
NW = NC * NS
ROWS_W = N // NW
CH = 128
NCH = ROWS_W // CH
NBUF = 4
L = 16
KV = D // L
EPS = 1e-12


def _ln_body(x_hbm, tab_hbm, pos_hbm, g_hbm, b_hbm, out_hbm,
             idx0, idx1, idx2, idx3, rows0, rows1, rows2, rows3,
             pos_v, g_v, b_v,
             gs0, gs1, gs2, gs3, ws0, ws1, ws2, ws3):
    idx_v = [idx0, idx1, idx2, idx3]
    rows_v = [rows0, rows1, rows2, rows3]
    gsem = [gs0, gs1, gs2, gs3]
    wsem = [ws0, ws1, ws2, ws3]

    wid = lax.axis_index("s") * NC + lax.axis_index("c")
    base_w = wid * ROWS_W

    pltpu.sync_copy(pos_hbm, pos_v)
    pltpu.sync_copy(g_hbm, g_v)
    pltpu.sync_copy(b_hbm, b_v)
    g = [g_v[pl.ds(k * L, L)] for k in range(KV)]
    bta = [b_v[pl.ds(k * L, L)] for k in range(KV)]

    lanes = lax.iota(jnp.int32, L)
    bfly = [jnp.bitwise_xor(lanes, sh).reshape(L, 1) for sh in (1, 2, 4, 8)]
    gdn = lax.GatherDimensionNumbers(
        offset_dims=(), collapsed_slice_dims=(0,), start_index_map=(0,))

    def allreduce_sum(v):
        for idx in bfly:
            v = v + lax.gather(v, idx, gdn, (1,),
                               mode=lax.GatherScatterMode.PROMISE_IN_BOUNDS)
        return v

    def start_gather(cc, b):
        pltpu.sync_copy(x_hbm.at[pl.ds(base_w + cc * CH, CH)], idx_v[b])
        pltpu.async_copy(tab_hbm.at[idx_v[b]], rows_v[b], gsem[b])

    def compute_chunk(cc, b):
        buf = rows_v[b]

        def row(r, rcarry):
            p = lax.rem(cc * CH + r, S)
            t = []
            for k in range(KV):
                tk = buf[r, pl.ds(k * L, L)] + pos_v[p, pl.ds(k * L, L)]
                t.append(tk)
            sq = [tk * tk for tk in t]
            acc = t
            acc2 = sq
            while len(acc) > 1:
                acc = [acc[i] + acc[i + 1] for i in range(0, len(acc), 2)]
                acc2 = [acc2[i] + acc2[i + 1] for i in range(0, len(acc2), 2)]
            mean = allreduce_sum(acc[0]) * (1.0 / D)
            ex2 = allreduce_sum(acc2[0]) * (1.0 / D)
            var = ex2 - mean * mean + EPS
            xi = lax.bitcast_convert_type(var, jnp.int32)
            yi = jnp.int32(0x5F3759DF) - lax.shift_right_logical(xi, 1)
            y = lax.bitcast_convert_type(yi, jnp.float32)
            h = var * 0.5
            for _ in range(2):
                y = y * (1.5 - h * y * y)
            for k in range(KV):
                u = (t[k] - mean) * y * g[k] + bta[k]
                buf[r, pl.ds(k * L, L)] = u
            return rcarry

        lax.fori_loop(0, CH, row, 0, unroll=4)

    for b in range(NBUF - 1):
        start_gather(b, b)

    def ring(i, carry):
        for b in range(NBUF):
            cc = i * NBUF + b
            pb = (b - 1) % NBUF
            pltpu.make_async_copy(rows_v[b], out_hbm.at[pl.ds(0, CH)],
                                  gsem[b]).wait()
            compute_chunk(cc, b)
            pltpu.async_copy(rows_v[b],
                             out_hbm.at[pl.ds(base_w + cc * CH, CH)], wsem[b])

            @pl.when(cc >= 1)
            def _wait_prev_write():
                pltpu.make_async_copy(
                    rows_v[pb], out_hbm.at[pl.ds(0, CH)], wsem[pb]).wait()

            @pl.when(cc + NBUF - 1 < NCH)
            def _prefetch():
                start_gather(cc + NBUF - 1, pb)
        return carry

    lax.fori_loop(0, NCH // NBUF, ring, 0)
    pltpu.make_async_copy(rows_v[(NCH - 1) % NBUF], out_hbm.at[pl.ds(0, CH)],
                          wsem[(NCH - 1) % NBUF]).wait()


def kernel(x, id_table, pos_table, ln_gamma, ln_beta):
    xf = x.reshape(N)
    mesh = plsc.VectorSubcoreMesh(core_axis_name="c", subcore_axis_name="s")
    run = pl.kernel(
        _ln_body,
        mesh=mesh,
        out_type=jax.ShapeDtypeStruct((N, D), jnp.float32),
        scratch_types=(
            [pltpu.VMEM((CH,), jnp.int32) for _ in range(NBUF)]
            + [pltpu.VMEM((CH, D), jnp.float32) for _ in range(NBUF)]
            + [pltpu.VMEM((S, D), jnp.float32),
               pltpu.VMEM((D,), jnp.float32),
               pltpu.VMEM((D,), jnp.float32)]
            + [pltpu.SemaphoreType.DMA for _ in range(2 * NBUF)]
        ),
    )
    out = run(xf, id_table, pos_table, ln_gamma, ln_beta)
    return out.reshape(B, S, D)


# speedup vs baseline: 5.0538x; 1.7568x over previous
import jax
import jax.numpy as jnp
from jax import lax
from jax.experimental import pallas as pl
from jax.experimental.pallas import tpu as pltpu
from jax.experimental.pallas import tpu_sc as plsc

B = 4096
S = 200
D = 128
N = B * S
NC = 2
NS = 16
NW = NC * NS
ROWS_W = N // NW
CH = 128
NCH = ROWS_W // CH
NBUF = 4
BB = 16
EPS = 1e-12


def _gather_body(x_hbm, tab_hbm, out_hbm,
                 idx0, idx1, idx2, idx3, rows0, rows1, rows2, rows3,
                 gs0, gs1, gs2, gs3, ws0, ws1, ws2, ws3):
    idx_v = [idx0, idx1, idx2, idx3]
    rows_v = [rows0, rows1, rows2, rows3]
    gsem = [gs0, gs1, gs2, gs3]
    wsem = [ws0, ws1, ws2, ws3]

    wid = lax.axis_index("s") * NC + lax.axis_index("c")
    base_w = wid * ROWS_W

    def start_gather(cc, b):
        pltpu.sync_copy(x_hbm.at[pl.ds(base_w + cc * CH, CH)], idx_v[b])
        pltpu.async_copy(tab_hbm.at[idx_v[b]], rows_v[b], gsem[b])

    for b in range(NBUF - 1):
        start_gather(b, b)

    def ring(i, carry):
        for b in range(NBUF):
            cc = i * NBUF + b
            pb = (b - 1) % NBUF
            pltpu.make_async_copy(rows_v[b], out_hbm.at[pl.ds(0, CH)],
                                  gsem[b]).wait()
            pltpu.async_copy(rows_v[b],
                             out_hbm.at[pl.ds(base_w + cc * CH, CH)], wsem[b])

            @pl.when(cc >= 1)
            def _wait_prev_write():
                pltpu.make_async_copy(
                    rows_v[pb], out_hbm.at[pl.ds(0, CH)], wsem[pb]).wait()

            @pl.when(cc + NBUF - 1 < NCH)
            def _prefetch():
                start_gather(cc + NBUF - 1, pb)
        return carry

    lax.fori_loop(0, NCH // NBUF, ring, 0)
    pltpu.make_async_copy(rows_v[(NCH - 1) % NBUF], out_hbm.at[pl.ds(0, CH)],
                          wsem[(NCH - 1) % NBUF]).wait()


def _sc_gather(xf, id_table):
    mesh = plsc.VectorSubcoreMesh(core_axis_name="c", subcore_axis_name="s")
    run = pl.kernel(
        _gather_body,
        mesh=mesh,
        out_type=jax.ShapeDtypeStruct((N, D), jnp.float32),
        scratch_types=(
            [pltpu.VMEM((CH,), jnp.int32) for _ in range(NBUF)]
            + [pltpu.VMEM((CH, D), jnp.float32) for _ in range(NBUF)]
            + [pltpu.SemaphoreType.DMA for _ in range(2 * NBUF)]
        ),
    )
    return run(xf, id_table)


def _ln_body(t_ref, pos_ref, g_ref, b_ref, o_ref):
    t = t_ref[...] + pos_ref[...]
    mean = jnp.mean(t, axis=-1, keepdims=True)
    var = jnp.mean(t * t, axis=-1, keepdims=True) - mean * mean
    y = lax.rsqrt(var + EPS)
    o_ref[...] = (t - mean) * y * g_ref[...] + b_ref[...]


def _tc_ln(t, pos_table, ln_gamma, ln_beta):
    pos3 = pos_table.reshape(1, S, D)
    g3 = ln_gamma.reshape(1, 1, D)
    b3 = ln_beta.reshape(1, 1, D)
    return pl.pallas_call(
        _ln_body,
        grid=(B // BB,),
        in_specs=[
            pl.BlockSpec((BB, S, D), lambda i: (i, 0, 0)),
            pl.BlockSpec((1, S, D), lambda i: (0, 0, 0)),
            pl.BlockSpec((1, 1, D), lambda i: (0, 0, 0)),
            pl.BlockSpec((1, 1, D), lambda i: (0, 0, 0)),
        ],
        out_specs=pl.BlockSpec((BB, S, D), lambda i: (i, 0, 0)),
        out_shape=jax.ShapeDtypeStruct((B, S, D), jnp.float32),
    )(t, pos3, g3, b3)


def kernel(x, id_table, pos_table, ln_gamma, ln_beta):
    rows = _sc_gather(x.reshape(N), id_table)
    return _tc_ln(rows.reshape(B, S, D), pos_table, ln_gamma, ln_beta)
